# baseline (speedup 1.0000x reference)
import functools

import jax
import jax.numpy as jnp
from jax import lax
from jax.experimental import pallas as pl
from jax.experimental.pallas import tpu as pltpu
from jax.experimental.pallas import tpu_sc as plsc

N_NODES = 100000
N_EDGES = 1600000

LANE = 128
G = 4
ROWS = 12800
EDGES_PAD = ROWS * LANE
RPT1 = ROWS // 16
RPT2 = ROWS // 32
ACC_R = 100352
ZCHUNKS = ACC_R // (16 * 128)
OUT_RPT = 6256
OUT_LAST = N_NODES - 15 * OUT_RPT
DUMMY_SPREAD = ACC_R - N_NODES
F32 = jnp.float32


def _agg_run(src_r, dst_r, tref, acc, srcv, dstv, gbuf, gsem, ssem, b0, rpt):
    ng = rpt // G

    def fire_gather(p, j):
        return pltpu.async_copy(tref.at[srcv.at[p, j]], gbuf.at[p, j],
                                gsem.at[p])

    def drain_gather(p, j):
        pltpu.make_async_copy(tref.at[srcv.at[p, j]], gbuf.at[p, j],
                              gsem.at[p]).wait()

    def fire_scatter(p, j):
        return pltpu.async_copy(gbuf.at[p, j], acc.at[dstv.at[p, j]],
                                ssem.at[p], add=True)

    def drain_scatter(p, j):
        pltpu.make_async_copy(gbuf.at[p, j], acc.at[dstv.at[p, j]],
                              ssem.at[p]).wait()

    def load_idx(p, row0):
        pltpu.sync_copy(src_r.at[pl.ds(row0, G), :], srcv.at[p])
        pltpu.sync_copy(dst_r.at[pl.ds(row0, G), :], dstv.at[p])

    load_idx(0, b0)
    for j in range(G):
        fire_gather(0, j)
    load_idx(1, b0 + G)
    for j in range(G):
        fire_gather(1, j)

    def step(g, r, t):
        b = r
        nxt = (r + 2) % 3

        def prefetch():
            load_idx(nxt, b0 + (g + 2) * G)
            for j in range(G):
                fire_gather(nxt, j)

        if r == 0:
            @pl.when(jnp.logical_and(g + 2 < ng, t >= 1))
            def _():
                for j in range(G):
                    drain_scatter(nxt, j)

            @pl.when(g + 2 < ng)
            def _():
                prefetch()
        else:
            @pl.when(g + 2 < ng)
            def _():
                for j in range(G):
                    drain_scatter(nxt, j)
                prefetch()

        for j in range(G):
            drain_gather(b, j)
        for j in range(G):
            fire_scatter(b, j)

    def triple_body(t, carry):
        for r in (0, 1, 2):
            step(t * 3 + r, r, t)
        return carry

    nt = ng // 3
    lax.fori_loop(0, nt, triple_body, 0)
    for g in range(nt * 3, ng):
        b = g % 3
        for j in range(G):
            drain_gather(b, j)
        for j in range(G):
            fire_scatter(b, j)
    for g in range(ng - 3, ng):
        for j in range(G):
            drain_scatter(g % 3, j)


def _zero_acc(acc, zbuf, s):
    def zrow(i, c):
        zbuf[i, :] = jnp.zeros((16,), F32)
        return c

    lax.fori_loop(0, LANE, zrow, 0)
    zbase = s * (ZCHUNKS * LANE)
    for z in range(ZCHUNKS):
        pltpu.sync_copy(zbuf, acc.at[pl.ds(zbase + z * LANE, LANE), :])


def _copy_out(acc, out_c0, out_c1, c, s):
    def cp(out, sl):
        pltpu.sync_copy(acc.at[sl, :], out.at[sl, :])

    for cc, out in ((0, out_c0), (1, out_c1)):
        @pl.when(jnp.logical_and(c == cc, s < 15))
        def _():
            cp(out, pl.ds(s * OUT_RPT, OUT_RPT))

        @pl.when(jnp.logical_and(c == cc, s == 15))
        def _():
            cp(out, pl.ds(15 * OUT_RPT, OUT_LAST))


def _sc1_body(src_r, dst_r, ta, tb, out_a, out_b,
              acc, zbuf, srcv, dstv, gbuf, gsem, ssem):
    c = lax.axis_index("c")
    s = lax.axis_index("s")
    _zero_acc(acc, zbuf, s)
    plsc.subcore_barrier()

    @pl.when(c == 0)
    def _():
        _agg_run(src_r, dst_r, ta, acc, srcv, dstv, gbuf, gsem, ssem,
                 s * RPT1, RPT1)

    @pl.when(c == 1)
    def _():
        _agg_run(src_r, dst_r, tb, acc, srcv, dstv, gbuf, gsem, ssem,
                 s * RPT1, RPT1)

    plsc.subcore_barrier()
    _copy_out(acc, out_a, out_b, c, s)


def _sc2_body(src_r, dst_r, t2, out0, out1,
              acc, zbuf, srcv, dstv, gbuf, gsem, ssem):
    c = lax.axis_index("c")
    s = lax.axis_index("s")
    _zero_acc(acc, zbuf, s)
    plsc.subcore_barrier()
    w = c * 16 + s
    _agg_run(src_r, dst_r, t2, acc, srcv, dstv, gbuf, gsem, ssem,
             w * RPT2, RPT2)
    plsc.subcore_barrier()
    _copy_out(acc, out0, out1, c, s)


def _sc_scratch():
    return [
        pltpu.VMEM_SHARED((ACC_R, 16), F32),
        pltpu.VMEM((LANE, 16), F32),
        pltpu.VMEM((3, G, LANE), jnp.int32),
        pltpu.VMEM((3, G, LANE), jnp.int32),
        pltpu.VMEM((3, G, LANE, 16), F32),
        pltpu.SemaphoreType.DMA((3,)),
        pltpu.SemaphoreType.DMA((3,)),
    ]


def _sc_mesh():
    return plsc.VectorSubcoreMesh(core_axis_name="c", subcore_axis_name="s")


_SC_PARAMS = pltpu.CompilerParams(use_tc_tiling_on_sc=False)


NPACK = N_NODES // 8
BLKP = 1280
GRIDP = (NPACK + BLKP - 1) // BLKP


def _tc1_body(x_ref, bta_ref, btb_ref, bra_ref, brb_ref, onesp_ref,
              b1a_ref, b1b_ref, ta_ref, tb_ref, ra_ref, rb_ref):
    xb = x_ref[...]
    ta_ref[...] = jnp.dot(xb, bta_ref[...], preferred_element_type=F32)
    tb_ref[...] = (jnp.dot(xb, btb_ref[...], preferred_element_type=F32)
                   + onesp_ref[...])
    ra_ref[...] = (jnp.dot(xb, bra_ref[...], preferred_element_type=F32)
                   + b1a_ref[...])
    rb_ref[...] = (jnp.dot(xb, brb_ref[...], preferred_element_type=F32)
                   + b1b_ref[...])


def _tc2_body(aa_ref, ab_ref, ra_ref, rb_ref, c1_ref, b2a_ref, b2b_ref,
              br2a_ref, br2b_ref, b2p_ref, t2_ref, r2_ref):
    aa = aa_ref[...]
    ab = ab_ref[...]
    cntb = jnp.maximum(jnp.dot(ab, c1_ref[...], preferred_element_type=F32),
                       1.0)
    h1a = jnp.maximum(aa / cntb + ra_ref[...], 0.0)
    h1b = jnp.maximum(ab / cntb + rb_ref[...], 0.0)
    t2_ref[...] = (jnp.dot(h1a, b2a_ref[...], preferred_element_type=F32)
                   + jnp.dot(h1b, b2b_ref[...], preferred_element_type=F32))
    r2_ref[...] = (jnp.dot(h1a, br2a_ref[...], preferred_element_type=F32)
                   + jnp.dot(h1b, br2b_ref[...], preferred_element_type=F32)
                   + b2p_ref[...])


def _tc3_body(p0_ref, p1_ref, ab_ref, c1_ref, r2_ref, out_ref):
    tot = p0_ref[...] + p1_ref[...]
    cntb = jnp.maximum(jnp.dot(ab_ref[...], c1_ref[...],
                               preferred_element_type=F32), 1.0)
    out_ref[...] = jnp.maximum(tot / cntb + r2_ref[...], 0.0)


def _row_spec(d):
    return pl.BlockSpec((BLKP, d), lambda i: (i, 0))


def _full_spec(*shape):
    nd = len(shape)
    return pl.BlockSpec(shape, lambda i: (0,) * nd)


def _pad_to(m, r, c):
    return jnp.pad(m, ((0, r - m.shape[0]), (0, c - m.shape[1])))


@jax.jit
def kernel(x, edge_index, W_l1, W_r1, b1, W_l2, W_r2, b2):
    f32 = F32
    src = edge_index[0]
    dst = edge_index[1]
    pad = EDGES_PAD - N_EDGES
    ar = jnp.arange(pad, dtype=jnp.int32)
    src_p = jnp.concatenate([src, ar % 4096]).reshape(ROWS, LANE)
    dst_p = jnp.concatenate([dst, N_NODES + ar % DUMMY_SPREAD]).reshape(ROWS, LANE)

    eye8 = jnp.eye(8, dtype=f32)
    ones8 = jnp.ones((8,), f32)
    kron = jnp.kron
    bd_ta = kron(eye8, _pad_to(W_l1[:, :16], 32, 16))
    bd_tb = kron(eye8, _pad_to(W_l1[:, 16:21], 32, 16))
    bd_ra = kron(eye8, _pad_to(W_r1[:, :16], 32, 16))
    bd_rb = kron(eye8, _pad_to(W_r1[:, 16:21], 32, 16))
    e5 = jnp.zeros((16,), f32).at[5].set(1.0)
    onesp = kron(ones8, e5)
    b1a = kron(ones8, b1[:16])
    b1b = kron(ones8, jnp.pad(b1[16:21], (0, 11)))
    c1 = kron(eye8, jnp.outer(e5, jnp.ones((16,), f32)))
    bd2a = kron(eye8, _pad_to(W_l2[:16, :], 16, 16))
    bd2b = kron(eye8, _pad_to(W_l2[16:21, :], 16, 16))
    bdr2a = kron(eye8, _pad_to(W_r2[:16, :], 16, 16))
    bdr2b = kron(eye8, _pad_to(W_r2[16:21, :], 16, 16))
    b2p = kron(ones8, jnp.pad(b2, (0, 6)))

    x8 = x.reshape(NPACK, 256)

    ta_p, tb_p, ra_p, rb_p = pl.pallas_call(
        _tc1_body,
        grid=(GRIDP,),
        in_specs=[_row_spec(256), _full_spec(256, 128), _full_spec(256, 128),
                  _full_spec(256, 128), _full_spec(256, 128),
                  _full_spec(128), _full_spec(128), _full_spec(128)],
        out_specs=[_row_spec(128)] * 4,
        out_shape=[jax.ShapeDtypeStruct((NPACK, 128), f32)] * 4,
    )(x8, bd_ta, bd_tb, bd_ra, bd_rb, onesp, b1a, b1b)

    acc_a, acc_b = pl.kernel(
        _sc1_body,
        out_type=[jax.ShapeDtypeStruct((N_NODES, 16), f32),
                  jax.ShapeDtypeStruct((N_NODES, 16), f32)],
        mesh=_sc_mesh(),
        scratch_types=_sc_scratch(),
        compiler_params=_SC_PARAMS,
    )(src_p, dst_p, ta_p.reshape(N_NODES, 16), tb_p.reshape(N_NODES, 16))

    t2_p, r2_p = pl.pallas_call(
        _tc2_body,
        grid=(GRIDP,),
        in_specs=[_row_spec(128)] * 4
        + [_full_spec(128, 128)] * 5 + [_full_spec(128)],
        out_specs=[_row_spec(128)] * 2,
        out_shape=[jax.ShapeDtypeStruct((NPACK, 128), f32)] * 2,
    )(acc_a.reshape(NPACK, 128), acc_b.reshape(NPACK, 128), ra_p, rb_p,
      c1, bd2a, bd2b, bdr2a, bdr2b, b2p)

    part0, part1 = pl.kernel(
        _sc2_body,
        out_type=[jax.ShapeDtypeStruct((N_NODES, 16), f32),
                  jax.ShapeDtypeStruct((N_NODES, 16), f32)],
        mesh=_sc_mesh(),
        scratch_types=_sc_scratch(),
        compiler_params=_SC_PARAMS,
    )(src_p, dst_p, t2_p.reshape(N_NODES, 16))

    out_p = pl.pallas_call(
        _tc3_body,
        grid=(GRIDP,),
        in_specs=[_row_spec(128)] * 3 + [_full_spec(128, 128), _row_spec(128)],
        out_specs=_row_spec(128),
        out_shape=jax.ShapeDtypeStruct((NPACK, 128), f32),
    )(part0.reshape(NPACK, 128), part1.reshape(NPACK, 128),
      acc_b.reshape(NPACK, 128), c1, r2_p)
    return out_p.reshape(N_NODES, 16)[:, :10]


# speedup vs baseline: 1.0900x; 1.0900x over previous
import functools

import jax
import jax.numpy as jnp
from jax import lax
from jax.experimental import pallas as pl
from jax.experimental.pallas import tpu as pltpu
from jax.experimental.pallas import tpu_sc as plsc

N_NODES = 100000
N_EDGES = 1600000

LANE = 128
G = 5
ROWS = 12800
EDGES_PAD = ROWS * LANE
RPT1 = ROWS // 16
RPT2 = ROWS // 32
ACC_R = 100352
ZCHUNKS = ACC_R // (16 * 128)
OUT_RPT = 6256
OUT_LAST = N_NODES - 15 * OUT_RPT
DUMMY_SPREAD = ACC_R - N_NODES
F32 = jnp.float32


def _agg_run(src_r, dst_r, tref, acc, srcv, dstv, gbuf, gsem, ssem, b0, rpt):
    ng = rpt // G

    def fire_gather(p, j):
        return pltpu.async_copy(tref.at[srcv.at[p, j]], gbuf.at[p, j],
                                gsem.at[p])

    def drain_gather(p, j):
        pltpu.make_async_copy(tref.at[srcv.at[p, j]], gbuf.at[p, j],
                              gsem.at[p]).wait()

    def fire_scatter(p, j):
        return pltpu.async_copy(gbuf.at[p, j], acc.at[dstv.at[p, j]],
                                ssem.at[p], add=True)

    def drain_scatter(p, j):
        pltpu.make_async_copy(gbuf.at[p, j], acc.at[dstv.at[p, j]],
                              ssem.at[p]).wait()

    def load_idx(p, row0):
        pltpu.sync_copy(src_r.at[pl.ds(row0, G), :], srcv.at[p])
        pltpu.sync_copy(dst_r.at[pl.ds(row0, G), :], dstv.at[p])

    load_idx(0, b0)
    for j in range(G):
        fire_gather(0, j)

    def pair_body(g2, carry):
        for p in (0, 1):
            g = g2 * 2 + p
            q = 1 - p

            @pl.when(g >= 1)
            def _():
                for j in range(G):
                    drain_scatter(q, j)

            @pl.when(g + 1 < ng)
            def _():
                load_idx(q, b0 + (g + 1) * G)
                for j in range(G):
                    fire_gather(q, j)

            for j in range(G):
                drain_gather(p, j)
            for j in range(G):
                fire_scatter(p, j)
        return carry

    lax.fori_loop(0, ng // 2, pair_body, 0)
    for j in range(G):
        drain_scatter((ng - 1) % 2, j)


def _zero_acc(acc, zbuf, s):
    def zrow(i, c):
        zbuf[i, :] = jnp.zeros((16,), F32)
        return c

    lax.fori_loop(0, LANE, zrow, 0)
    zbase = s * (ZCHUNKS * LANE)
    for z in range(ZCHUNKS):
        pltpu.sync_copy(zbuf, acc.at[pl.ds(zbase + z * LANE, LANE), :])


def _copy_out(acc, out_c0, out_c1, c, s):
    def cp(out, sl):
        pltpu.sync_copy(acc.at[sl, :], out.at[sl, :])

    for cc, out in ((0, out_c0), (1, out_c1)):
        @pl.when(jnp.logical_and(c == cc, s < 15))
        def _():
            cp(out, pl.ds(s * OUT_RPT, OUT_RPT))

        @pl.when(jnp.logical_and(c == cc, s == 15))
        def _():
            cp(out, pl.ds(15 * OUT_RPT, OUT_LAST))


def _sc1_body(src_r, dst_r, ta, tb, out_a, out_b,
              acc, zbuf, srcv, dstv, gbuf, gsem, ssem):
    c = lax.axis_index("c")
    s = lax.axis_index("s")
    _zero_acc(acc, zbuf, s)
    plsc.subcore_barrier()

    @pl.when(c == 0)
    def _():
        _agg_run(src_r, dst_r, ta, acc, srcv, dstv, gbuf, gsem, ssem,
                 s * RPT1, RPT1)

    @pl.when(c == 1)
    def _():
        _agg_run(src_r, dst_r, tb, acc, srcv, dstv, gbuf, gsem, ssem,
                 s * RPT1, RPT1)

    plsc.subcore_barrier()
    _copy_out(acc, out_a, out_b, c, s)


def _sc2_body(src_r, dst_r, t2, out0, out1,
              acc, zbuf, srcv, dstv, gbuf, gsem, ssem):
    c = lax.axis_index("c")
    s = lax.axis_index("s")
    _zero_acc(acc, zbuf, s)
    plsc.subcore_barrier()
    w = c * 16 + s
    _agg_run(src_r, dst_r, t2, acc, srcv, dstv, gbuf, gsem, ssem,
             w * RPT2, RPT2)
    plsc.subcore_barrier()
    _copy_out(acc, out0, out1, c, s)


def _sc_scratch():
    return [
        pltpu.VMEM_SHARED((ACC_R, 16), F32),
        pltpu.VMEM((LANE, 16), F32),
        pltpu.VMEM((2, G, LANE), jnp.int32),
        pltpu.VMEM((2, G, LANE), jnp.int32),
        pltpu.VMEM((2, G, LANE, 16), F32),
        pltpu.SemaphoreType.DMA((2,)),
        pltpu.SemaphoreType.DMA((2,)),
    ]


def _sc_mesh():
    return plsc.VectorSubcoreMesh(core_axis_name="c", subcore_axis_name="s")


_SC_PARAMS = pltpu.CompilerParams(use_tc_tiling_on_sc=False)


NPACK = N_NODES // 8
BLKP = 1280
GRIDP = (NPACK + BLKP - 1) // BLKP


def _tc1_body(x_ref, bta_ref, btb_ref, bra_ref, brb_ref, onesp_ref,
              b1a_ref, b1b_ref, ta_ref, tb_ref, ra_ref, rb_ref):
    xb = x_ref[...]
    ta_ref[...] = jnp.dot(xb, bta_ref[...], preferred_element_type=F32)
    tb_ref[...] = (jnp.dot(xb, btb_ref[...], preferred_element_type=F32)
                   + onesp_ref[...])
    ra_ref[...] = (jnp.dot(xb, bra_ref[...], preferred_element_type=F32)
                   + b1a_ref[...])
    rb_ref[...] = (jnp.dot(xb, brb_ref[...], preferred_element_type=F32)
                   + b1b_ref[...])


def _tc2_body(aa_ref, ab_ref, ra_ref, rb_ref, c1_ref, b2a_ref, b2b_ref,
              br2a_ref, br2b_ref, b2p_ref, t2_ref, r2_ref):
    aa = aa_ref[...]
    ab = ab_ref[...]
    cntb = jnp.maximum(jnp.dot(ab, c1_ref[...], preferred_element_type=F32),
                       1.0)
    h1a = jnp.maximum(aa / cntb + ra_ref[...], 0.0)
    h1b = jnp.maximum(ab / cntb + rb_ref[...], 0.0)
    t2_ref[...] = (jnp.dot(h1a, b2a_ref[...], preferred_element_type=F32)
                   + jnp.dot(h1b, b2b_ref[...], preferred_element_type=F32))
    r2_ref[...] = (jnp.dot(h1a, br2a_ref[...], preferred_element_type=F32)
                   + jnp.dot(h1b, br2b_ref[...], preferred_element_type=F32)
                   + b2p_ref[...])


def _tc3_body(p0_ref, p1_ref, ab_ref, c1_ref, r2_ref, out_ref):
    tot = p0_ref[...] + p1_ref[...]
    cntb = jnp.maximum(jnp.dot(ab_ref[...], c1_ref[...],
                               preferred_element_type=F32), 1.0)
    out_ref[...] = jnp.maximum(tot / cntb + r2_ref[...], 0.0)


def _row_spec(d):
    return pl.BlockSpec((BLKP, d), lambda i: (i, 0))


def _full_spec(*shape):
    nd = len(shape)
    return pl.BlockSpec(shape, lambda i: (0,) * nd)


def _pad_to(m, r, c):
    return jnp.pad(m, ((0, r - m.shape[0]), (0, c - m.shape[1])))


@jax.jit
def kernel(x, edge_index, W_l1, W_r1, b1, W_l2, W_r2, b2):
    f32 = F32
    src = edge_index[0]
    dst = edge_index[1]
    pad = EDGES_PAD - N_EDGES
    ar = jnp.arange(pad, dtype=jnp.int32)
    src_p = jnp.concatenate([src, ar % 4096]).reshape(ROWS, LANE)
    dst_p = jnp.concatenate([dst, N_NODES + ar % DUMMY_SPREAD]).reshape(ROWS, LANE)

    eye8 = jnp.eye(8, dtype=f32)
    ones8 = jnp.ones((8,), f32)
    kron = jnp.kron
    bd_ta = kron(eye8, _pad_to(W_l1[:, :16], 32, 16))
    bd_tb = kron(eye8, _pad_to(W_l1[:, 16:21], 32, 16))
    bd_ra = kron(eye8, _pad_to(W_r1[:, :16], 32, 16))
    bd_rb = kron(eye8, _pad_to(W_r1[:, 16:21], 32, 16))
    e5 = jnp.zeros((16,), f32).at[5].set(1.0)
    onesp = kron(ones8, e5)
    b1a = kron(ones8, b1[:16])
    b1b = kron(ones8, jnp.pad(b1[16:21], (0, 11)))
    c1 = kron(eye8, jnp.outer(e5, jnp.ones((16,), f32)))
    bd2a = kron(eye8, _pad_to(W_l2[:16, :], 16, 16))
    bd2b = kron(eye8, _pad_to(W_l2[16:21, :], 16, 16))
    bdr2a = kron(eye8, _pad_to(W_r2[:16, :], 16, 16))
    bdr2b = kron(eye8, _pad_to(W_r2[16:21, :], 16, 16))
    b2p = kron(ones8, jnp.pad(b2, (0, 6)))

    x8 = x.reshape(NPACK, 256)

    ta_p, tb_p, ra_p, rb_p = pl.pallas_call(
        _tc1_body,
        grid=(GRIDP,),
        in_specs=[_row_spec(256), _full_spec(256, 128), _full_spec(256, 128),
                  _full_spec(256, 128), _full_spec(256, 128),
                  _full_spec(128), _full_spec(128), _full_spec(128)],
        out_specs=[_row_spec(128)] * 4,
        out_shape=[jax.ShapeDtypeStruct((NPACK, 128), f32)] * 4,
    )(x8, bd_ta, bd_tb, bd_ra, bd_rb, onesp, b1a, b1b)

    acc_a, acc_b = pl.kernel(
        _sc1_body,
        out_type=[jax.ShapeDtypeStruct((N_NODES, 16), f32),
                  jax.ShapeDtypeStruct((N_NODES, 16), f32)],
        mesh=_sc_mesh(),
        scratch_types=_sc_scratch(),
        compiler_params=_SC_PARAMS,
    )(src_p, dst_p, ta_p.reshape(N_NODES, 16), tb_p.reshape(N_NODES, 16))

    t2_p, r2_p = pl.pallas_call(
        _tc2_body,
        grid=(GRIDP,),
        in_specs=[_row_spec(128)] * 4
        + [_full_spec(128, 128)] * 5 + [_full_spec(128)],
        out_specs=[_row_spec(128)] * 2,
        out_shape=[jax.ShapeDtypeStruct((NPACK, 128), f32)] * 2,
    )(acc_a.reshape(NPACK, 128), acc_b.reshape(NPACK, 128), ra_p, rb_p,
      c1, bd2a, bd2b, bdr2a, bdr2b, b2p)

    part0, part1 = pl.kernel(
        _sc2_body,
        out_type=[jax.ShapeDtypeStruct((N_NODES, 16), f32),
                  jax.ShapeDtypeStruct((N_NODES, 16), f32)],
        mesh=_sc_mesh(),
        scratch_types=_sc_scratch(),
        compiler_params=_SC_PARAMS,
    )(src_p, dst_p, t2_p.reshape(N_NODES, 16))

    out_p = pl.pallas_call(
        _tc3_body,
        grid=(GRIDP,),
        in_specs=[_row_spec(128)] * 3 + [_full_spec(128, 128), _row_spec(128)],
        out_specs=_row_spec(128),
        out_shape=jax.ShapeDtypeStruct((NPACK, 128), f32),
    )(part0.reshape(NPACK, 128), part1.reshape(NPACK, 128),
      acc_b.reshape(NPACK, 128), c1, r2_p)
    return out_p.reshape(N_NODES, 16)[:, :10]
